# Initial kernel scaffold; baseline (speedup 1.0000x reference)
import jax
import jax.numpy as jnp
from jax.experimental import pallas as pl


def kernel(channel_ids, table):
    raise NotImplementedError("write your pallas kernel here")


# speedup vs baseline: 4.9354x; 4.9354x over previous
import functools

import jax
import jax.numpy as jnp
from jax import lax
from jax.experimental import pallas as pl
from jax.experimental.pallas import tpu as pltpu
from jax.experimental.pallas import tpu_sc as plsc

_NUM_CHANNELS = 1000000
_D = 32
_BATCH = 16384
_HIST = 200
_N = _BATCH * _HIST
_IW = 128
_NROWS = _N // _IW
_NC = 2
_NS = 16
_NW = _NC * _NS
_RPW = _NROWS // _NW
_G = 16
_CHUNKS = _RPW // _G
_CROWS = _G * _IW


def _sc_gather(ids2d, table):
    mesh = plsc.VectorSubcoreMesh(
        core_axis_name="c", subcore_axis_name="s",
        num_cores=_NC, num_subcores=_NS)

    @functools.partial(
        pl.kernel,
        out_type=jax.ShapeDtypeStruct((_N, _D), jnp.float32),
        mesh=mesh,
        scratch_types=[
            pltpu.VMEM((_G, _IW), jnp.int32),
            pltpu.VMEM((_CROWS, _D), jnp.float32),
            pltpu.SemaphoreType.DMA,
        ],
        compiler_params=pltpu.CompilerParams(use_tc_tiling_on_sc=False),
    )
    def k(idx_hbm, table_hbm, out_hbm, idx_v, rows_v, sem):
        wid = lax.axis_index("s") * _NC + lax.axis_index("c")
        row0 = wid * _RPW

        @pl.loop(0, _CHUNKS)
        def _chunk(c):
            rbase = row0 + c * _G
            pltpu.sync_copy(idx_hbm.at[pl.ds(rbase, _G)], idx_v)

            def _clamp_row(j, _):
                def _clamp16(t, _):
                    v = idx_v[j, pl.ds(t * 16, 16)]
                    v = jnp.minimum(jnp.maximum(v, 0), _NUM_CHANNELS - 1)
                    idx_v[j, pl.ds(t * 16, 16)] = v
                    return 0
                return lax.fori_loop(0, _IW // 16, _clamp16, 0)

            lax.fori_loop(0, _G, _clamp_row, 0)

            copies = [
                pltpu.async_copy(
                    table_hbm.at[idx_v.at[j]],
                    rows_v.at[pl.ds(j * _IW, _IW)],
                    sem,
                )
                for j in range(_G)
            ]
            for cp in copies:
                cp.wait()
            pltpu.sync_copy(rows_v, out_hbm.at[pl.ds(rbase * _IW, _CROWS)])

    return k(ids2d, table)


def kernel(channel_ids, table):
    ids2d = channel_ids.astype(jnp.int32).reshape(_NROWS, _IW)
    out = _sc_gather(ids2d, table)
    return out.reshape(_BATCH, _HIST, _D)
